# Initial kernel scaffold; baseline (speedup 1.0000x reference)
import jax
import jax.numpy as jnp
from jax.experimental import pallas as pl


def kernel(input_feat, adj_indices, adj_values, weight, bias):
    raise NotImplementedError("write your pallas kernel here")


# speedup vs baseline: 2.7037x; 2.7037x over previous
import functools

import jax
import jax.numpy as jnp
from jax import lax
from jax.experimental import pallas as pl
from jax.experimental.pallas import tpu as pltpu
from jax.experimental.pallas import tpu_sc as plsc

N = 10000
E = 160000
D = 256
HALF = 128

NC = 2
NS = 16
L = 16

CH = 128
EPT = E // NS
FULL = EPT // CH
REM = EPT - FULL * CH

STRIPE = 632
ROWS_PAD = NS * STRIPE


def _sc_spmm(x2, row, col, val):
  mesh = plsc.VectorSubcoreMesh(
      core_axis_name="c", subcore_axis_name="s", num_cores=NC,
      num_subcores=NS)

  @functools.partial(
      pl.kernel,
      out_type=jax.ShapeDtypeStruct((NC, N, HALF), jnp.float32),
      mesh=mesh,
      scratch_types=[
          pltpu.VMEM((CH,), jnp.int32),
          pltpu.VMEM((1, CH), jnp.int32),
          pltpu.VMEM((1, CH), jnp.int32),
          pltpu.VMEM((CH,), jnp.float32),
          pltpu.VMEM((CH, HALF), jnp.float32),
          pltpu.VMEM((CH, HALF), jnp.float32),
          pltpu.VMEM_SHARED((ROWS_PAD, HALF), jnp.float32),
          pltpu.SemaphoreType.DMA,
          pltpu.SemaphoreType.DMA,
      ],
  )
  def k(x2_hbm, row_hbm, col_hbm, val_hbm, out_hbm,
        colv, gidx, ridx, valv, rowsv, zerov, accum, gsem, ssem):
    c = lax.axis_index("c")
    s = lax.axis_index("s")

    zvec = jnp.zeros((L,), jnp.float32)

    def zrow(i, carry):
      for kk in range(HALF // L):
        zerov[i, pl.ds(kk * L, L)] = zvec
      return carry

    lax.fori_loop(0, CH, zrow, 0)
    base = s * STRIPE
    for j in range(STRIPE // CH):
      pltpu.sync_copy(zerov, accum.at[pl.ds(base + j * CH, CH)])
    rem_rows = STRIPE - (STRIPE // CH) * CH
    pltpu.sync_copy(zerov.at[pl.ds(0, rem_rows)],
                    accum.at[pl.ds(base + (STRIPE // CH) * CH, rem_rows)])
    plsc.subcore_barrier()

    def do_chunk(ebase, nreal):
      pltpu.sync_copy(col_hbm.at[pl.ds(ebase, nreal)],
                      colv.at[pl.ds(0, nreal)])
      pltpu.sync_copy(row_hbm.at[pl.ds(ebase, nreal)],
                      ridx.at[0].at[pl.ds(0, nreal)])
      pltpu.sync_copy(val_hbm.at[pl.ds(ebase, nreal)],
                      valv.at[pl.ds(0, nreal)])
      for kk in range(nreal // L):
        cv = colv[pl.ds(kk * L, L)]
        gidx[0, pl.ds(kk * L, L)] = cv * 2 + c
      if nreal < CH:
        for kk in range(nreal // L, CH // L):
          gidx[0, pl.ds(kk * L, L)] = jnp.zeros((L,), jnp.int32)
          ridx[0, pl.ds(kk * L, L)] = jnp.full((L,), N, jnp.int32)
      pltpu.async_copy(x2_hbm.at[gidx.at[0]], rowsv, gsem).wait()

      def scale(g, carry):
        vv = valv[pl.ds(g * L, L)]
        for j in range(L):
          e = g * L + j
          bv = jnp.full((L,), vv[j], jnp.float32)
          for kk in range(HALF // L):
            rowsv[e, pl.ds(kk * L, L)] = rowsv[e, pl.ds(kk * L, L)] * bv
        return carry

      lax.fori_loop(0, nreal // L, scale, 0)
      pltpu.async_copy(rowsv, accum.at[ridx.at[0]], ssem, add=True).wait()

    def chunk_body(j, carry):
      do_chunk(s * EPT + j * CH, CH)
      return carry

    lax.fori_loop(0, FULL, chunk_body, 0)
    if REM:
      do_chunk(s * EPT + FULL * CH, REM)

    plsc.subcore_barrier()
    last = N - (NS - 1) * STRIPE

    @pl.when(s < NS - 1)
    def _():
      pltpu.sync_copy(accum.at[pl.ds(base, STRIPE)],
                      out_hbm.at[c, pl.ds(base, STRIPE)])

    @pl.when(s == NS - 1)
    def _():
      pltpu.sync_copy(accum.at[pl.ds(base, last)],
                      out_hbm.at[c, pl.ds(base, last)])

  return k(x2, row, col, val)


def _tc_matmul(agg2, w2, b):
  BM = 1000

  def mm(a_ref, w_ref, b_ref, o_ref):
    acc = jnp.dot(a_ref[0], w_ref[0], preferred_element_type=jnp.float32)
    acc = acc + jnp.dot(a_ref[1], w_ref[1],
                        preferred_element_type=jnp.float32)
    o_ref[...] = acc + b_ref[...]

  return pl.pallas_call(
      mm,
      grid=(N // BM,),
      in_specs=[
          pl.BlockSpec((NC, BM, HALF), lambda i: (0, i, 0)),
          pl.BlockSpec((NC, HALF, D), lambda i: (0, 0, 0)),
          pl.BlockSpec((1, D), lambda i: (0, 0)),
      ],
      out_specs=pl.BlockSpec((BM, D), lambda i: (i, 0)),
      out_shape=jax.ShapeDtypeStruct((N, D), jnp.float32),
  )(agg2, w2, b)


@jax.jit
def kernel(input_feat, adj_indices, adj_values, weight, bias):
  x2 = input_feat.reshape(2 * N, HALF)
  row = adj_indices[0].astype(jnp.int32)
  col = adj_indices[1].astype(jnp.int32)
  agg2 = _sc_spmm(x2, row, col, adj_values)
  return _tc_matmul(agg2, weight.reshape(NC, HALF, D),
                    bias.reshape(1, D))


# speedup vs baseline: 3.6210x; 1.3392x over previous
import functools

import jax
import jax.numpy as jnp
from jax import lax
from jax.experimental import pallas as pl
from jax.experimental.pallas import tpu as pltpu
from jax.experimental.pallas import tpu_sc as plsc

N = 10000
E = 160000
D = 256
HALF = 128

NC = 2
NS = 16
L = 16

CH = 128
CPT = 80
BLK = 16
NBLK = CPT // BLK
EPAD = NS * CPT * CH

STRIPE = 632
ROWS_PAD = NS * STRIPE
LAST = N - (NS - 1) * STRIPE


def _sc_spmm(x2, gcol, row3d, val3d):
  mesh = plsc.VectorSubcoreMesh(
      core_axis_name="c", subcore_axis_name="s", num_cores=NC,
      num_subcores=NS)

  @functools.partial(
      pl.kernel,
      out_type=jax.ShapeDtypeStruct((NC, N, HALF), jnp.float32),
      mesh=mesh,
      scratch_types=[
          pltpu.VMEM((2, BLK, CH), jnp.int32),
          pltpu.VMEM((2, BLK, CH), jnp.int32),
          pltpu.VMEM((2, BLK, CH), jnp.float32),
          pltpu.VMEM((CH, HALF), jnp.float32),
          pltpu.VMEM((CH, HALF), jnp.float32),
          pltpu.VMEM_SHARED((ROWS_PAD, HALF), jnp.float32),
          pltpu.SemaphoreType.DMA,
          pltpu.SemaphoreType.DMA,
          pltpu.SemaphoreType.DMA,
      ],
  )
  def k(x2_hbm, gcol_hbm, row3d_hbm, val3d_hbm, out_hbm,
        gidxv, ridxv, valv, rows0, rows1, accum, gsem0, gsem1, bsem):
    c = lax.axis_index("c")
    s = lax.axis_index("s")

    zvec = jnp.zeros((L,), jnp.float32)

    def zrow(i, carry):
      for kk in range(HALF // L):
        rows0[i, pl.ds(kk * L, L)] = zvec
      return carry

    lax.fori_loop(0, CH, zrow, 0)
    base = s * STRIPE
    for j in range(STRIPE // CH):
      pltpu.sync_copy(rows0, accum.at[pl.ds(base + j * CH, CH)])
    rem_rows = STRIPE - (STRIPE // CH) * CH
    pltpu.sync_copy(rows0.at[pl.ds(0, rem_rows)],
                    accum.at[pl.ds(base + (STRIPE // CH) * CH, rem_rows)])
    plsc.subcore_barrier()

    def issue_blk(t, tb):
      pltpu.async_copy(gcol_hbm.at[c, s, pl.ds(t * BLK, BLK)],
                       gidxv.at[tb], bsem)
      pltpu.async_copy(row3d_hbm.at[s, pl.ds(t * BLK, BLK)],
                       ridxv.at[tb], bsem)
      pltpu.async_copy(val3d_hbm.at[s, pl.ds(t * BLK, BLK)],
                       valv.at[tb], bsem)

    def wait_blk(t, tb):
      pltpu.make_async_copy(gcol_hbm.at[c, s, pl.ds(t * BLK, BLK)],
                            gidxv.at[tb], bsem).wait()
      pltpu.make_async_copy(row3d_hbm.at[s, pl.ds(t * BLK, BLK)],
                            ridxv.at[tb], bsem).wait()
      pltpu.make_async_copy(val3d_hbm.at[s, pl.ds(t * BLK, BLK)],
                            valv.at[tb], bsem).wait()

    issue_blk(0, 0)
    wait_blk(0, 0)

    pltpu.async_copy(x2_hbm.at[gidxv.at[0, 0]], rows0, gsem0)

    bufs = (rows0, rows1)
    sems = (gsem0, gsem1)

    def process(tb, jj, b):
      cur = bufs[b]

      pltpu.make_async_copy(x2_hbm.at[gidxv.at[tb, jj]], cur,
                            sems[b]).wait()

      def scale(g, carry):
        vv = valv[tb, jj, pl.ds(g * L, L)]
        for lane in range(L):
          e = g * L + lane
          bv = jnp.full((L,), vv[lane], jnp.float32)
          for kk in range(HALF // L):
            cur[e, pl.ds(kk * L, L)] = cur[e, pl.ds(kk * L, L)] * bv
        return carry

      lax.fori_loop(0, CH // L, scale, 0)
      pltpu.sync_copy(cur, accum.at[ridxv.at[tb, jj]], add=True)

    def prefetch(tb, jj, b):
      pltpu.async_copy(x2_hbm.at[gidxv.at[tb, jj]], bufs[b], sems[b])

    for t in range(NBLK):
      tb = t & 1
      if t + 1 < NBLK:
        issue_blk(t + 1, 1 - tb)

      def inner(g, carry, t=t, tb=tb):
        prefetch(tb, 2 * g + 1, 1)
        process(tb, 2 * g, 0)
        if t + 1 < NBLK:
          @pl.when(g < BLK // 2 - 1)
          def _():
            prefetch(tb, 2 * g + 2, 0)

          @pl.when(g == BLK // 2 - 1)
          def _():
            wait_blk(t + 1, 1 - tb)
            prefetch(1 - tb, 0, 0)
        else:
          @pl.when(g < BLK // 2 - 1)
          def _():
            prefetch(tb, 2 * g + 2, 0)
        process(tb, 2 * g + 1, 1)
        return carry

      lax.fori_loop(0, BLK // 2, inner, 0)

    plsc.subcore_barrier()

    @pl.when(s < NS - 1)
    def _():
      pltpu.sync_copy(accum.at[pl.ds(base, STRIPE)],
                      out_hbm.at[c, pl.ds(base, STRIPE)])

    @pl.when(s == NS - 1)
    def _():
      pltpu.sync_copy(accum.at[pl.ds(base, LAST)],
                      out_hbm.at[c, pl.ds(base, LAST)])

  return k(x2, gcol, row3d, val3d)


def _tc_matmul(agg2, w2, b):
  BM = 1000

  def mm(a_ref, w_ref, b_ref, o_ref):
    acc = jnp.dot(a_ref[0], w_ref[0], preferred_element_type=jnp.float32)
    acc = acc + jnp.dot(a_ref[1], w_ref[1],
                        preferred_element_type=jnp.float32)
    o_ref[...] = acc + b_ref[...]

  return pl.pallas_call(
      mm,
      grid=(N // BM,),
      in_specs=[
          pl.BlockSpec((NC, BM, HALF), lambda i: (0, i, 0)),
          pl.BlockSpec((NC, HALF, D), lambda i: (0, 0, 0)),
          pl.BlockSpec((1, D), lambda i: (0, 0)),
      ],
      out_specs=pl.BlockSpec((BM, D), lambda i: (i, 0)),
      out_shape=jax.ShapeDtypeStruct((N, D), jnp.float32),
  )(agg2, w2, b)


@jax.jit
def kernel(input_feat, adj_indices, adj_values, weight, bias):
  x2 = input_feat.reshape(2 * N, HALF)
  row = adj_indices[0].astype(jnp.int32)
  col = adj_indices[1].astype(jnp.int32)
  pad = EPAD - E
  rowp = jnp.concatenate([row, jnp.full((pad,), N, jnp.int32)])
  colp = jnp.concatenate([col, jnp.zeros((pad,), jnp.int32)])
  valp = jnp.concatenate([adj_values, jnp.zeros((pad,), jnp.float32)])
  gcol = jnp.stack([colp * 2, colp * 2 + 1]).reshape(NC, NS, CPT, CH)
  row3d = rowp.reshape(NS, CPT, CH)
  val3d = valp.reshape(NS, CPT, CH)
  agg2 = _sc_spmm(x2, gcol, row3d, val3d)
  return _tc_matmul(agg2, weight.reshape(NC, HALF, D),
                    bias.reshape(1, D))
